# Initial kernel scaffold; baseline (speedup 1.0000x reference)
import jax
import jax.numpy as jnp
from jax.experimental import pallas as pl


def kernel(x, edge_index, edge_attr, W_e0, b_e0, W_e1, b_e1, W_g0, b_g0, W_g1, b_g1, W_n0, b_n0, W_n1, b_n1, W_c, b_c, edge_indices, edge_attrs, edge_indices_f2c, position, node_attrs, clusters):
    raise NotImplementedError("write your pallas kernel here")


# speedup vs baseline: 5.2301x; 5.2301x over previous
import functools

import jax
import jax.numpy as jnp
from jax import lax
from jax.experimental import pallas as pl
from jax.experimental.pallas import tpu as pltpu
from jax.experimental.pallas import tpu_sc as plsc

N = 10000
E = 320000
H = 128
NPAD = 10240
NR = NPAD // 128
NC = 2
NS = 16
NW = NC * NS
BPT = 80
K = 128
CB = 8
NQ = BPT // CB
EPAD = NW * BPT * K
RPT = NR // NS
SPT = NPAD // NS

f32 = jnp.float32
i32 = jnp.int32


def _elu(v):
    return jnp.where(v > 0, v, jnp.exp(jnp.minimum(v, 0.0)) - 1.0)


def _dot(a, b):
    return lax.dot_general(a, b, (((1,), (0,)), ((), ())),
                           preferred_element_type=f32)


def _edge_mlp_body(attr_ref, we0_ref, be0_ref, we1_ref, be1_ref, out_ref):
    t = attr_ref[...]
    hcol = _elu(t * we0_ref[...] + be0_ref[...])
    out_ref[...] = _elu(_dot(hcol, we1_ref[...]) + be1_ref[...])


def _edge_mlp(edge_attr, W_e0, b_e0, W_e1, b_e1):
    BE = 8000
    return pl.pallas_call(
        _edge_mlp_body,
        grid=(E // BE,),
        in_specs=[
            pl.BlockSpec((BE, 1), lambda i: (i, 0)),
            pl.BlockSpec((1, H), lambda i: (0, 0)),
            pl.BlockSpec((1, H), lambda i: (0, 0)),
            pl.BlockSpec((H, 1), lambda i: (0, 0)),
            pl.BlockSpec((1, 1), lambda i: (0, 0)),
        ],
        out_specs=pl.BlockSpec((BE, 1), lambda i: (i, 0)),
        out_shape=jax.ShapeDtypeStruct((E, 1), f32),
    )(edge_attr, W_e0, b_e0.reshape(1, H), W_e1, b_e1.reshape(1, 1))


_MESH = plsc.VectorSubcoreMesh(core_axis_name="c", subcore_axis_name="s")
_PARAMS = pltpu.CompilerParams(needs_layout_passes=False)

_Z16F = functools.partial(jnp.zeros, (16,), f32)


def _splat(vec, r):
    idxs = jnp.full((16,), r, i32)
    dn = lax.GatherDimensionNumbers(offset_dims=(), collapsed_slice_dims=(0,),
                                    start_index_map=(0,))
    return lax.gather(vec, idxs[:, None], dn, (1,),
                      mode=lax.GatherScatterMode.PROMISE_IN_BOUNDS)


def _zero_gbuf(gbuf):
    def bz(r, _):
        for g in range(8):
            gbuf[r, pl.ds(g * 16, 16)] = _Z16F()
        return 0
    lax.fori_loop(0, K, bz, 0)


def _zero_shagg(sh_agg, gbuf, s):
    for j in range(SPT // K):
        pltpu.sync_copy(gbuf, sh_agg.at[pl.ds(s * SPT + j * K, K)])


def _scale_block(gbuf, norm_ref, b):
    def brow(r16, _):
        nv = norm_ref[b, pl.ds(r16 * 16, 16)]
        for r in range(16):
            spl = _splat(nv, r)
            row = r16 * 16 + r
            for g in range(8):
                sl = pl.ds(g * 16, 16)
                gbuf[row, sl] = gbuf[row, sl] * spl
        return 0
    lax.fori_loop(0, K // 16, brow, 0)


def _agg_chunk(tab_hbm, cs_src, cs_dst, cs_norm, gbuf, gsem, sh_agg):
    for b in range(CB):
        pltpu.async_copy(tab_hbm.at[cs_src.at[b]], gbuf, gsem).wait()
        _scale_block(gbuf, cs_norm, b)
        pltpu.sync_copy(gbuf, sh_agg.at[cs_dst.at[b]], add=True)


def _out_copy(agg_h, sh_agg, c, s):
    for j in range(SPT // K):
        sl = pl.ds(s * SPT + j * K, K)
        pltpu.sync_copy(sh_agg.at[sl], agg_h.at[c, sl])


def _sc1_body(x_hbm, srcp_h, dstp_h, eap_h,
              dinv_h, normp_h, agg_h,
              cs_src, cs_dst, cs_ea, cs_norm,
              dacc, dtmp, dinv_sl, dinv_v, rows_v, gbuf, gsem,
              sh_deg, sh_dinv, sh_agg):
    c = lax.axis_index("c")
    s = lax.axis_index("s")
    wid = s * NC + c

    _zero_gbuf(gbuf)
    _zero_shagg(sh_agg, gbuf, s)

    def bz(r, _):
        for g in range(8):
            dacc[r, pl.ds(g * 16, 16)] = _Z16F()
        return 0
    lax.fori_loop(0, NR, bz, 0)

    @pl.when(s == 0)
    def _():
        pltpu.sync_copy(dacc, sh_deg)

    def bi(i, _):
        rows_v[0, pl.ds(i * 16, 16)] = lax.iota(i32, 16) + i * 16
        return 0
    lax.fori_loop(0, NR // 16, bi, 0)

    for qq in range(2):
        slab = s * 2 + qq

        def bq(q, _):
            ci = slab * NQ + q
            pltpu.sync_copy(srcp_h.at[ci], cs_src)
            pltpu.sync_copy(dstp_h.at[ci], cs_dst)
            pltpu.sync_copy(eap_h.at[ci], cs_ea)

            def bd(i, _):
                b = i >> 3
                off = (i & 7) * 16
                sv = cs_src[b, pl.ds(off, 16)]
                dv = cs_dst[b, pl.ds(off, 16)]
                ev = cs_ea[b, pl.ds(off, 16)]
                wmv = jnp.where(sv != dv, ev, _Z16F())
                plsc.addupdate_scatter(
                    dacc,
                    [lax.shift_right_logical(dv, 7),
                     jnp.bitwise_and(dv, 127)],
                    wmv)
                return 0
            lax.fori_loop(0, CB * 8, bd, 0)
            return 0
        lax.fori_loop(0, NQ, bq, 0)

    plsc.subcore_barrier()

    pltpu.sync_copy(dacc, sh_deg.at[rows_v.at[0]], add=True)
    plsc.subcore_barrier()

    pltpu.sync_copy(sh_deg.at[pl.ds(s * RPT, RPT)], dtmp)
    for r in range(RPT):
        for g in range(8):
            sl = pl.ds(g * 16, 16)
            xv = dtmp[r, sl] + 1.0
            ib = lax.bitcast_convert_type(xv, i32)
            ib = jnp.int32(0x5F3759DF) - lax.shift_right_logical(ib, 1)
            y = lax.bitcast_convert_type(ib, f32)
            for _u in range(3):
                y = y * (1.5 - 0.5 * xv * y * y)
            y = jnp.where(xv > 0.0, y,
                          jnp.where(xv == 0.0, _Z16F(),
                                    jnp.full((16,), jnp.nan, f32)))
            dinv_sl[r, sl] = y

    pltpu.sync_copy(dinv_sl, sh_dinv.at[pl.ds(s * RPT, RPT)])
    plsc.subcore_barrier()
    pltpu.sync_copy(sh_dinv, dinv_v)

    @pl.when(c == 0)
    def _():
        pltpu.sync_copy(dinv_sl, dinv_h.at[s])

    def bq2(q, _):
        ci = wid * NQ + q
        pltpu.sync_copy(srcp_h.at[ci], cs_src)
        pltpu.sync_copy(dstp_h.at[ci], cs_dst)
        pltpu.sync_copy(eap_h.at[ci], cs_ea)

        def bn(i, _):
            b = i >> 3
            off = (i & 7) * 16
            sv = cs_src[b, pl.ds(off, 16)]
            dv = cs_dst[b, pl.ds(off, 16)]
            ev = cs_ea[b, pl.ds(off, 16)]
            wmv = jnp.where(sv != dv, ev, _Z16F())
            gs = plsc.load_gather(
                dinv_v, [lax.shift_right_logical(sv, 7),
                         jnp.bitwise_and(sv, 127)])
            gd = plsc.load_gather(
                dinv_v, [lax.shift_right_logical(dv, 7),
                         jnp.bitwise_and(dv, 127)])
            cs_norm[b, pl.ds(off, 16)] = gs * wmv * gd
            return 0
        lax.fori_loop(0, CB * 8, bn, 0)

        pltpu.sync_copy(cs_norm, normp_h.at[ci])
        _agg_chunk(x_hbm, cs_src, cs_dst, cs_norm, gbuf, gsem, sh_agg)
        return 0
    lax.fori_loop(0, NQ, bq2, 0)

    plsc.subcore_barrier()
    _out_copy(agg_h, sh_agg, c, s)


def _sc_conv1(x, srcp, dstp, eap):
    fn = pl.kernel(
        _sc1_body,
        mesh=_MESH,
        compiler_params=_PARAMS,
        out_type=(
            jax.ShapeDtypeStruct((NS, RPT, 128), f32),
            jax.ShapeDtypeStruct((NW * NQ, CB, K), f32),
            jax.ShapeDtypeStruct((NC, NPAD, H), f32),
        ),
        scratch_types=[
            pltpu.VMEM((CB, K), i32),
            pltpu.VMEM((CB, K), i32),
            pltpu.VMEM((CB, K), f32),
            pltpu.VMEM((CB, K), f32),
            pltpu.VMEM((NR, 128), f32),
            pltpu.VMEM((RPT, 128), f32),
            pltpu.VMEM((RPT, 128), f32),
            pltpu.VMEM((NR, 128), f32),
            pltpu.VMEM((1, NR), i32),
            pltpu.VMEM((K, H), f32),
            pltpu.SemaphoreType.DMA,
            pltpu.VMEM_SHARED((NR, 128), f32),
            pltpu.VMEM_SHARED((NR, 128), f32),
            pltpu.VMEM_SHARED((NPAD, H), f32),
        ],
    )
    return fn(x, srcp, dstp, eap)


def _sc2_body(h_hbm, srcp_h, dstp_h, normp_h, agg_h,
              cs_src, cs_dst, cs_norm, gbuf, gsem, sh_agg):
    c = lax.axis_index("c")
    s = lax.axis_index("s")
    wid = s * NC + c

    _zero_gbuf(gbuf)
    _zero_shagg(sh_agg, gbuf, s)
    plsc.subcore_barrier()

    def bq(q, _):
        ci = wid * NQ + q
        pltpu.sync_copy(srcp_h.at[ci], cs_src)
        pltpu.sync_copy(dstp_h.at[ci], cs_dst)
        pltpu.sync_copy(normp_h.at[ci], cs_norm)
        _agg_chunk(h_hbm, cs_src, cs_dst, cs_norm, gbuf, gsem, sh_agg)
        return 0
    lax.fori_loop(0, NQ, bq, 0)

    plsc.subcore_barrier()
    _out_copy(agg_h, sh_agg, c, s)


def _sc_conv2(h1, srcp, dstp, normp):
    fn = pl.kernel(
        _sc2_body,
        mesh=_MESH,
        compiler_params=_PARAMS,
        out_type=jax.ShapeDtypeStruct((NC, NPAD, H), f32),
        scratch_types=[
            pltpu.VMEM((CB, K), i32),
            pltpu.VMEM((CB, K), i32),
            pltpu.VMEM((CB, K), f32),
            pltpu.VMEM((K, H), f32),
            pltpu.SemaphoreType.DMA,
            pltpu.VMEM_SHARED((NPAD, H), f32),
        ],
    )
    return fn(h1, srcp, dstp, normp)


def _layer_body(agga_ref, aggb_ref, xin_ref, dinv_ref, w_ref, b_ref, out_ref):
    d = dinv_ref[...]
    z = agga_ref[...] + aggb_ref[...] + (d * d) * xin_ref[...]
    out_ref[...] = _elu(_dot(z, w_ref[...]) + b_ref[...])


def _gcn_layer(agga, aggb, xin, dinv_col, W, b):
    BN = 2000
    return pl.pallas_call(
        _layer_body,
        grid=(N // BN,),
        in_specs=[
            pl.BlockSpec((BN, H), lambda i: (i, 0)),
            pl.BlockSpec((BN, H), lambda i: (i, 0)),
            pl.BlockSpec((BN, H), lambda i: (i, 0)),
            pl.BlockSpec((BN, 1), lambda i: (i, 0)),
            pl.BlockSpec((H, H), lambda i: (0, 0)),
            pl.BlockSpec((1, H), lambda i: (0, 0)),
        ],
        out_specs=pl.BlockSpec((BN, H), lambda i: (i, 0)),
        out_shape=jax.ShapeDtypeStruct((N, H), f32),
    )(agga, aggb, xin, dinv_col, W, b.reshape(1, H))


def _final_body(agga_ref, aggb_ref, hin_ref, dinv_ref, wg_ref, bg_ref,
                wn0_ref, bn0_ref, wn1_ref, bn1_ref, wc_ref, bc_ref, out_ref):
    d = dinv_ref[...]
    z = agga_ref[...] + aggb_ref[...] + (d * d) * hin_ref[...]
    h2 = _elu(_dot(z, wg_ref[...]) + bg_ref[...])
    n1 = _elu(_dot(h2, wn0_ref[...]) + bn0_ref[...])
    v = _elu(_dot(n1, wn1_ref[...]) + bn1_ref[...])
    idx = lax.broadcasted_iota(i32, (N, 1), 0)
    vm = jnp.where(idx == 0, 0.0, pltpu.roll(v, 1, 0))
    vp = jnp.where(idx == N - 1, 0.0, pltpu.roll(v, N - 1, 0))
    k0 = wc_ref[0, 0]
    k1 = wc_ref[0, 1]
    k2 = wc_ref[0, 2]
    out_ref[...] = k0 * vm + k1 * v + k2 * vp + bc_ref[0, 0]


def _final(agga, aggb, h1, dinv_col, W_g1, b_g1, W_n0, b_n0, W_n1, b_n1,
           wc2d, bc2d):
    return pl.pallas_call(
        _final_body,
        grid=(1,),
        in_specs=[
            pl.BlockSpec((N, H), lambda i: (0, 0)),
            pl.BlockSpec((N, H), lambda i: (0, 0)),
            pl.BlockSpec((N, H), lambda i: (0, 0)),
            pl.BlockSpec((N, 1), lambda i: (0, 0)),
            pl.BlockSpec((H, H), lambda i: (0, 0)),
            pl.BlockSpec((1, H), lambda i: (0, 0)),
            pl.BlockSpec((H, H), lambda i: (0, 0)),
            pl.BlockSpec((1, H), lambda i: (0, 0)),
            pl.BlockSpec((H, 1), lambda i: (0, 0)),
            pl.BlockSpec((1, 1), lambda i: (0, 0)),
            pl.BlockSpec(memory_space=pltpu.SMEM),
            pl.BlockSpec(memory_space=pltpu.SMEM),
        ],
        out_specs=pl.BlockSpec((N, 1), lambda i: (0, 0)),
        out_shape=jax.ShapeDtypeStruct((N, 1), f32),
    )(agga, aggb, h1, dinv_col, W_g1, b_g1.reshape(1, H),
      W_n0, b_n0.reshape(1, H), W_n1, b_n1.reshape(1, 1), wc2d, bc2d)


def kernel(x, edge_index, edge_attr, W_e0, b_e0, W_e1, b_e1, W_g0, b_g0,
           W_g1, b_g1, W_n0, b_n0, W_n1, b_n1, W_c, b_c, edge_indices=None,
           edge_attrs=None, edge_indices_f2c=None, position=None,
           node_attrs=None, clusters=None):
    x = x.astype(f32)
    ei = edge_index.astype(i32)
    ea = _edge_mlp(edge_attr.astype(f32), W_e0, b_e0, W_e1, b_e1)

    pad = EPAD - E
    srcp = jnp.pad(ei[0], (0, pad)).reshape(NW * NQ, CB, K)
    dstp = jnp.pad(ei[1], (0, pad)).reshape(NW * NQ, CB, K)
    eap = jnp.pad(ea[:, 0], (0, pad)).reshape(NW * NQ, CB, K)

    dinv2d, normp, agg0 = _sc_conv1(x, srcp, dstp, eap)
    dinv_col = dinv2d.reshape(NPAD)[:N][:, None]

    h1 = _gcn_layer(agg0[0, :N], agg0[1, :N], x, dinv_col, W_g0, b_g0)
    agg1 = _sc_conv2(h1, srcp, dstp, normp)
    out = _final(agg1[0, :N], agg1[1, :N], h1, dinv_col, W_g1, b_g1,
                 W_n0, b_n0, W_n1, b_n1, W_c.reshape(1, 3), b_c.reshape(1, 1))
    return (out, edge_index, ea)


# speedup vs baseline: 6.2194x; 1.1892x over previous
import functools

import jax
import jax.numpy as jnp
from jax import lax
from jax.experimental import pallas as pl
from jax.experimental.pallas import tpu as pltpu
from jax.experimental.pallas import tpu_sc as plsc

N = 10000
E = 320000
H = 128
NPAD = 10240
NR = NPAD // 128
NC = 2
NS = 16
NW = NC * NS
BPT = 160
K = 64
CB = 16
NQ = BPT // CB
EPAD = NW * BPT * K
RPT = NR // NS
SPT = NPAD // NS

f32 = jnp.float32
i32 = jnp.int32


def _elu(v):
    return jnp.where(v > 0, v, jnp.exp(jnp.minimum(v, 0.0)) - 1.0)


def _dot(a, b):
    return lax.dot_general(a, b, (((1,), (0,)), ((), ())),
                           preferred_element_type=f32)


def _edge_mlp_body(attr_ref, we0_ref, be0_ref, we1_ref, be1_ref, out_ref):
    t = attr_ref[...]
    hcol = _elu(t * we0_ref[...] + be0_ref[...])
    out_ref[...] = _elu(_dot(hcol, we1_ref[...]) + be1_ref[...])


def _edge_mlp(edge_attr, W_e0, b_e0, W_e1, b_e1):
    BE = 8000
    return pl.pallas_call(
        _edge_mlp_body,
        grid=(E // BE,),
        in_specs=[
            pl.BlockSpec((BE, 1), lambda i: (i, 0)),
            pl.BlockSpec((1, H), lambda i: (0, 0)),
            pl.BlockSpec((1, H), lambda i: (0, 0)),
            pl.BlockSpec((H, 1), lambda i: (0, 0)),
            pl.BlockSpec((1, 1), lambda i: (0, 0)),
        ],
        out_specs=pl.BlockSpec((BE, 1), lambda i: (i, 0)),
        out_shape=jax.ShapeDtypeStruct((E, 1), f32),
    )(edge_attr, W_e0, b_e0.reshape(1, H), W_e1, b_e1.reshape(1, 1))


_MESH = plsc.VectorSubcoreMesh(core_axis_name="c", subcore_axis_name="s")
_PARAMS = pltpu.CompilerParams(needs_layout_passes=False)

_Z16F = functools.partial(jnp.zeros, (16,), f32)


def _splat(vec, r):
    idxs = jnp.full((16,), r, i32)
    dn = lax.GatherDimensionNumbers(offset_dims=(), collapsed_slice_dims=(0,),
                                    start_index_map=(0,))
    return lax.gather(vec, idxs[:, None], dn, (1,),
                      mode=lax.GatherScatterMode.PROMISE_IN_BOUNDS)


def _zero_gbuf(gbuf):
    def bz(r, _):
        for g in range(8):
            gbuf[r, pl.ds(g * 16, 16)] = _Z16F()
        return 0
    lax.fori_loop(0, K, bz, 0)


def _zero_shagg(sh_agg, gbuf, s):
    for j in range(SPT // K):
        pltpu.sync_copy(gbuf, sh_agg.at[pl.ds(s * SPT + j * K, K)])


def _scale_block(gbuf, norm_ref, b):
    def brow(r16, _):
        nv = norm_ref[b, pl.ds(r16 * 16, 16)]
        for r in range(16):
            spl = _splat(nv, r)
            row = r16 * 16 + r
            for g in range(8):
                sl = pl.ds(g * 16, 16)
                gbuf[row, sl] = gbuf[row, sl] * spl
        return 0
    lax.fori_loop(0, K // 16, brow, 0)


def _agg_chunk(tab_hbm, cs_src, cs_dst, cs_norm, bufs, gsems, ssems,
               sh_agg):
    scat = [None, None]
    gath = [None, None]
    gath[0] = pltpu.async_copy(tab_hbm.at[cs_src.at[0]], bufs[0], gsems[0])
    for b in range(CB):
        p = b % 2
        q = (b + 1) % 2
        if b + 1 < CB:
            if scat[q] is not None:
                scat[q].wait()
            gath[q] = pltpu.async_copy(tab_hbm.at[cs_src.at[b + 1]],
                                       bufs[q], gsems[q])
        gath[p].wait()
        _scale_block(bufs[p], cs_norm, b)
        scat[p] = pltpu.async_copy(bufs[p], sh_agg.at[cs_dst.at[b]],
                                   ssems[p], add=True)
    scat[0].wait()
    scat[1].wait()


def _out_copy(agg_h, sh_agg, c, s):
    for j in range(SPT // K):
        sl = pl.ds(s * SPT + j * K, K)
        pltpu.sync_copy(sh_agg.at[sl], agg_h.at[c, sl])


def _sc1_body(x_hbm, srcp_h, dstp_h, eap_h,
              dinv_h, normp_h, agg_h,
              cs_src, cs_dst, cs_ea, cs_norm,
              dacc, dtmp, dinv_sl, dinv_v, rows_v, gbuf, gbuf1,
              gsem0, gsem1, ssem0, ssem1,
              sh_deg, sh_dinv, sh_agg):
    c = lax.axis_index("c")
    s = lax.axis_index("s")
    wid = s * NC + c

    _zero_gbuf(gbuf)
    _zero_shagg(sh_agg, gbuf, s)

    def bz(r, _):
        for g in range(8):
            dacc[r, pl.ds(g * 16, 16)] = _Z16F()
        return 0
    lax.fori_loop(0, NR, bz, 0)

    @pl.when(s == 0)
    def _():
        pltpu.sync_copy(dacc, sh_deg)

    def bi(i, _):
        rows_v[0, pl.ds(i * 16, 16)] = lax.iota(i32, 16) + i * 16
        return 0
    lax.fori_loop(0, NR // 16, bi, 0)

    for qq in range(2):
        slab = s * 2 + qq

        def bq(q, _):
            ci = slab * NQ + q
            d1 = pltpu.async_copy(srcp_h.at[ci], cs_src, gsem0)
            d2 = pltpu.async_copy(dstp_h.at[ci], cs_dst, gsem1)
            d3 = pltpu.async_copy(eap_h.at[ci], cs_ea, ssem0)
            d1.wait(); d2.wait(); d3.wait()

            def bd(i, _):
                b = i >> 2
                off = (i & 3) * 16
                sv = cs_src[b, pl.ds(off, 16)]
                dv = cs_dst[b, pl.ds(off, 16)]
                ev = cs_ea[b, pl.ds(off, 16)]
                wmv = jnp.where(sv != dv, ev, _Z16F())
                plsc.addupdate_scatter(
                    dacc,
                    [lax.shift_right_logical(dv, 7),
                     jnp.bitwise_and(dv, 127)],
                    wmv)
                return 0
            lax.fori_loop(0, CB * (K // 16), bd, 0)
            return 0
        lax.fori_loop(0, NQ, bq, 0)

    plsc.subcore_barrier()

    pltpu.sync_copy(dacc, sh_deg.at[rows_v.at[0]], add=True)
    plsc.subcore_barrier()

    pltpu.sync_copy(sh_deg.at[pl.ds(s * RPT, RPT)], dtmp)
    for r in range(RPT):
        for g in range(8):
            sl = pl.ds(g * 16, 16)
            xv = dtmp[r, sl] + 1.0
            ib = lax.bitcast_convert_type(xv, i32)
            ib = jnp.int32(0x5F3759DF) - lax.shift_right_logical(ib, 1)
            y = lax.bitcast_convert_type(ib, f32)
            for _u in range(3):
                y = y * (1.5 - 0.5 * xv * y * y)
            y = jnp.where(xv > 0.0, y,
                          jnp.where(xv == 0.0, _Z16F(),
                                    jnp.full((16,), jnp.nan, f32)))
            dinv_sl[r, sl] = y

    pltpu.sync_copy(dinv_sl, sh_dinv.at[pl.ds(s * RPT, RPT)])
    plsc.subcore_barrier()
    pltpu.sync_copy(sh_dinv, dinv_v)

    @pl.when(c == 0)
    def _():
        pltpu.sync_copy(dinv_sl, dinv_h.at[s])

    def bq2(q, _):
        ci = wid * NQ + q
        d1 = pltpu.async_copy(srcp_h.at[ci], cs_src, gsem0)
        d2 = pltpu.async_copy(dstp_h.at[ci], cs_dst, gsem1)
        d3 = pltpu.async_copy(eap_h.at[ci], cs_ea, ssem0)
        d1.wait(); d2.wait(); d3.wait()

        def bn(i, _):
            b = i >> 2
            off = (i & 3) * 16
            sv = cs_src[b, pl.ds(off, 16)]
            dv = cs_dst[b, pl.ds(off, 16)]
            ev = cs_ea[b, pl.ds(off, 16)]
            wmv = jnp.where(sv != dv, ev, _Z16F())
            gs = plsc.load_gather(
                dinv_v, [lax.shift_right_logical(sv, 7),
                         jnp.bitwise_and(sv, 127)])
            gd = plsc.load_gather(
                dinv_v, [lax.shift_right_logical(dv, 7),
                         jnp.bitwise_and(dv, 127)])
            cs_norm[b, pl.ds(off, 16)] = gs * wmv * gd
            return 0
        lax.fori_loop(0, CB * (K // 16), bn, 0)

        pltpu.sync_copy(cs_norm, normp_h.at[ci])
        _agg_chunk(x_hbm, cs_src, cs_dst, cs_norm, (gbuf, gbuf1),
                   (gsem0, gsem1), (ssem0, ssem1), sh_agg)
        return 0
    lax.fori_loop(0, NQ, bq2, 0)

    plsc.subcore_barrier()
    _out_copy(agg_h, sh_agg, c, s)


def _sc_conv1(x, srcp, dstp, eap):
    fn = pl.kernel(
        _sc1_body,
        mesh=_MESH,
        compiler_params=_PARAMS,
        out_type=(
            jax.ShapeDtypeStruct((NS, RPT, 128), f32),
            jax.ShapeDtypeStruct((NW * NQ, CB, K), f32),
            jax.ShapeDtypeStruct((NC, NPAD, H), f32),
        ),
        scratch_types=[
            pltpu.VMEM((CB, K), i32),
            pltpu.VMEM((CB, K), i32),
            pltpu.VMEM((CB, K), f32),
            pltpu.VMEM((CB, K), f32),
            pltpu.VMEM((NR, 128), f32),
            pltpu.VMEM((RPT, 128), f32),
            pltpu.VMEM((RPT, 128), f32),
            pltpu.VMEM((NR, 128), f32),
            pltpu.VMEM((1, NR), i32),
            pltpu.VMEM((K, H), f32),
            pltpu.VMEM((K, H), f32),
            pltpu.SemaphoreType.DMA,
            pltpu.SemaphoreType.DMA,
            pltpu.SemaphoreType.DMA,
            pltpu.SemaphoreType.DMA,
            pltpu.VMEM_SHARED((NR, 128), f32),
            pltpu.VMEM_SHARED((NR, 128), f32),
            pltpu.VMEM_SHARED((NPAD, H), f32),
        ],
    )
    return fn(x, srcp, dstp, eap)


def _sc2_body(h_hbm, srcp_h, dstp_h, normp_h, agg_h,
              cs_src, cs_dst, cs_norm, gbuf, gbuf1,
              gsem0, gsem1, ssem0, ssem1, sh_agg):
    c = lax.axis_index("c")
    s = lax.axis_index("s")
    wid = s * NC + c

    _zero_gbuf(gbuf)
    _zero_shagg(sh_agg, gbuf, s)
    plsc.subcore_barrier()

    def bq(q, _):
        ci = wid * NQ + q
        d1 = pltpu.async_copy(srcp_h.at[ci], cs_src, gsem0)
        d2 = pltpu.async_copy(dstp_h.at[ci], cs_dst, gsem1)
        d3 = pltpu.async_copy(normp_h.at[ci], cs_norm, ssem0)
        d1.wait(); d2.wait(); d3.wait()
        _agg_chunk(h_hbm, cs_src, cs_dst, cs_norm, (gbuf, gbuf1),
                   (gsem0, gsem1), (ssem0, ssem1), sh_agg)
        return 0
    lax.fori_loop(0, NQ, bq, 0)

    plsc.subcore_barrier()
    _out_copy(agg_h, sh_agg, c, s)


def _sc_conv2(h1, srcp, dstp, normp):
    fn = pl.kernel(
        _sc2_body,
        mesh=_MESH,
        compiler_params=_PARAMS,
        out_type=jax.ShapeDtypeStruct((NC, NPAD, H), f32),
        scratch_types=[
            pltpu.VMEM((CB, K), i32),
            pltpu.VMEM((CB, K), i32),
            pltpu.VMEM((CB, K), f32),
            pltpu.VMEM((K, H), f32),
            pltpu.VMEM((K, H), f32),
            pltpu.SemaphoreType.DMA,
            pltpu.SemaphoreType.DMA,
            pltpu.SemaphoreType.DMA,
            pltpu.SemaphoreType.DMA,
            pltpu.VMEM_SHARED((NPAD, H), f32),
        ],
    )
    return fn(h1, srcp, dstp, normp)


def _layer_body(agga_ref, aggb_ref, xin_ref, dinv_ref, w_ref, b_ref, out_ref):
    d = dinv_ref[...]
    z = agga_ref[...] + aggb_ref[...] + (d * d) * xin_ref[...]
    out_ref[...] = _elu(_dot(z, w_ref[...]) + b_ref[...])


def _gcn_layer(agga, aggb, xin, dinv_col, W, b):
    BN = 2000
    return pl.pallas_call(
        _layer_body,
        grid=(N // BN,),
        in_specs=[
            pl.BlockSpec((BN, H), lambda i: (i, 0)),
            pl.BlockSpec((BN, H), lambda i: (i, 0)),
            pl.BlockSpec((BN, H), lambda i: (i, 0)),
            pl.BlockSpec((BN, 1), lambda i: (i, 0)),
            pl.BlockSpec((H, H), lambda i: (0, 0)),
            pl.BlockSpec((1, H), lambda i: (0, 0)),
        ],
        out_specs=pl.BlockSpec((BN, H), lambda i: (i, 0)),
        out_shape=jax.ShapeDtypeStruct((N, H), f32),
    )(agga, aggb, xin, dinv_col, W, b.reshape(1, H))


def _final_body(agga_ref, aggb_ref, hin_ref, dinv_ref, wg_ref, bg_ref,
                wn0_ref, bn0_ref, wn1_ref, bn1_ref, wc_ref, bc_ref, out_ref):
    d = dinv_ref[...]
    z = agga_ref[...] + aggb_ref[...] + (d * d) * hin_ref[...]
    h2 = _elu(_dot(z, wg_ref[...]) + bg_ref[...])
    n1 = _elu(_dot(h2, wn0_ref[...]) + bn0_ref[...])
    v = _elu(_dot(n1, wn1_ref[...]) + bn1_ref[...])
    idx = lax.broadcasted_iota(i32, (N, 1), 0)
    vm = jnp.where(idx == 0, 0.0, pltpu.roll(v, 1, 0))
    vp = jnp.where(idx == N - 1, 0.0, pltpu.roll(v, N - 1, 0))
    k0 = wc_ref[0, 0]
    k1 = wc_ref[0, 1]
    k2 = wc_ref[0, 2]
    out_ref[...] = k0 * vm + k1 * v + k2 * vp + bc_ref[0, 0]


def _final(agga, aggb, h1, dinv_col, W_g1, b_g1, W_n0, b_n0, W_n1, b_n1,
           wc2d, bc2d):
    return pl.pallas_call(
        _final_body,
        grid=(1,),
        in_specs=[
            pl.BlockSpec((N, H), lambda i: (0, 0)),
            pl.BlockSpec((N, H), lambda i: (0, 0)),
            pl.BlockSpec((N, H), lambda i: (0, 0)),
            pl.BlockSpec((N, 1), lambda i: (0, 0)),
            pl.BlockSpec((H, H), lambda i: (0, 0)),
            pl.BlockSpec((1, H), lambda i: (0, 0)),
            pl.BlockSpec((H, H), lambda i: (0, 0)),
            pl.BlockSpec((1, H), lambda i: (0, 0)),
            pl.BlockSpec((H, 1), lambda i: (0, 0)),
            pl.BlockSpec((1, 1), lambda i: (0, 0)),
            pl.BlockSpec(memory_space=pltpu.SMEM),
            pl.BlockSpec(memory_space=pltpu.SMEM),
        ],
        out_specs=pl.BlockSpec((N, 1), lambda i: (0, 0)),
        out_shape=jax.ShapeDtypeStruct((N, 1), f32),
    )(agga, aggb, h1, dinv_col, W_g1, b_g1.reshape(1, H),
      W_n0, b_n0.reshape(1, H), W_n1, b_n1.reshape(1, 1), wc2d, bc2d)


def kernel(x, edge_index, edge_attr, W_e0, b_e0, W_e1, b_e1, W_g0, b_g0,
           W_g1, b_g1, W_n0, b_n0, W_n1, b_n1, W_c, b_c, edge_indices=None,
           edge_attrs=None, edge_indices_f2c=None, position=None,
           node_attrs=None, clusters=None):
    x = x.astype(f32)
    ei = edge_index.astype(i32)
    ea = _edge_mlp(edge_attr.astype(f32), W_e0, b_e0, W_e1, b_e1)

    pad = EPAD - E
    srcp = jnp.pad(ei[0], (0, pad)).reshape(NW * NQ, CB, K)
    dstp = jnp.pad(ei[1], (0, pad)).reshape(NW * NQ, CB, K)
    eap = jnp.pad(ea[:, 0], (0, pad)).reshape(NW * NQ, CB, K)

    dinv2d, normp, agg0 = _sc_conv1(x, srcp, dstp, eap)
    dinv_col = dinv2d.reshape(NPAD)[:N][:, None]

    h1 = _gcn_layer(agg0[0, :N], agg0[1, :N], x, dinv_col, W_g0, b_g0)
    agg1 = _sc_conv2(h1, srcp, dstp, normp)
    out = _final(agg1[0, :N], agg1[1, :N], h1, dinv_col, W_g1, b_g1,
                 W_n0, b_n0, W_n1, b_n1, W_c.reshape(1, 3), b_c.reshape(1, 1))
    return (out, edge_index, ea)
